# Initial kernel scaffold; baseline (speedup 1.0000x reference)
import jax
import jax.numpy as jnp
from jax.experimental import pallas as pl


def kernel(x, emb):
    raise NotImplementedError("write your pallas kernel here")


# speedup vs baseline: 1.9252x; 1.9252x over previous
import jax
import jax.numpy as jnp
from jax.experimental import pallas as pl


def _add_body(x_ref, emb_ref, o_ref):
    o_ref[...] = x_ref[...] + emb_ref[...][None]


def kernel(x, emb):
    B, S, D = x.shape
    BS = 512
    grid = (S // BS, B)
    return pl.pallas_call(
        _add_body,
        grid=grid,
        in_specs=[
            pl.BlockSpec((1, BS, D), lambda i, j: (j, i, 0)),
            pl.BlockSpec((BS, D), lambda i, j: (i, 0)),
        ],
        out_specs=pl.BlockSpec((1, BS, D), lambda i, j: (j, i, 0)),
        out_shape=jax.ShapeDtypeStruct((B, S, D), x.dtype),
    )(x, emb)
